# baseline (speedup 1.0000x reference)
import functools

import jax
import jax.numpy as jnp
from jax import lax
from jax.experimental import pallas as pl
from jax.experimental.pallas import tpu as pltpu
from jax.experimental.pallas import tpu_sc as plsc

NUM_CODEBOOKS = 4
CODEBOOK_BITS = 17
CODEBOOK_DIM = 16
HIDDEN_SIZE = 768
B, S = 4, 4096
T = B * S
ROWS_PER_128 = 128 // CODEBOOK_DIM

_NC, _NS = 2, 16
_NW = _NC * _NS
_B_PER_W = T // _NW
_CHUNK = 128
_NCHUNK = _B_PER_W // _CHUNK
_L = 16


@functools.cache
def _gather_sc():
    @functools.partial(
        pl.kernel,
        mesh=plsc.VectorSubcoreMesh(core_axis_name="c", subcore_axis_name="s"),
        compiler_params=pltpu.CompilerParams(use_tc_tiling_on_sc=False),
        out_type=jax.ShapeDtypeStruct((T, 128), jnp.float32),
        scratch_types=[
            pltpu.VMEM((_NCHUNK, _CHUNK), jnp.int32),
            pltpu.VMEM((_B_PER_W, 128), jnp.float32),
            pltpu.SemaphoreType.DMA,
        ],
    )
    def gather(ids_hbm, table_hbm, out_hbm, idx_v, rows_v, sem):
        wid = lax.axis_index("s") * _NC + lax.axis_index("c")
        pltpu.sync_copy(ids_hbm.at[pl.ds(wid * _NCHUNK, _NCHUNK)], idx_v)
        for j in range(_NCHUNK):
            for k in range(_CHUNK // _L):
                s = pl.ds(k * _L, _L)
                idx_v[j, s] = lax.shift_right_logical(idx_v[j, s], 3)
        copies = [
            pltpu.async_copy(
                table_hbm.at[idx_v.at[j]],
                rows_v.at[pl.ds(j * _CHUNK, _CHUNK)],
                sem,
            )
            for j in range(_NCHUNK)
        ]
        for cp in copies:
            cp.wait()
        pltpu.sync_copy(rows_v, out_hbm.at[pl.ds(wid * _B_PER_W, _B_PER_W)])

    return gather


_TT = 2048
_NT = T // _TT


def _mlp_body(e_ref, id_ref, w1_ref, w2_ref, b2_ref, o_ref):
    e128 = e_ref[...]
    ids = id_ref[0]
    ids = ids.reshape(_TT, 1)
    sub = jnp.bitwise_and(ids, ROWS_PER_128 - 1)
    c = lax.shift_right_logical(ids, CODEBOOK_BITS)

    oh8 = (sub == lax.broadcasted_iota(jnp.int32, (1, ROWS_PER_128), 1))
    oh8 = oh8.astype(jnp.float32)
    e = None
    for k in range(ROWS_PER_128):
        part = e128[:, k * CODEBOOK_DIM:(k + 1) * CODEBOOK_DIM]
        part = part * oh8[:, k:k + 1]
        e = part if e is None else e + part

    oh4 = (c == lax.broadcasted_iota(jnp.int32, (1, NUM_CODEBOOKS), 1))
    oh4 = oh4.astype(jnp.float32)
    eb = e.astype(jnp.bfloat16)
    oh4b = oh4.astype(jnp.bfloat16)
    placed = jnp.concatenate(
        [eb * oh4b[:, i:i + 1] for i in range(NUM_CODEBOOKS)] + [oh4b],
        axis=1,
    )
    h = lax.dot_general(
        placed, w1_ref[...], (((1,), (0,)), ((), ())),
        preferred_element_type=jnp.float32,
    )

    g = 0.5 * h * (1.0 + lax.erf(h * 0.7071067811865476))
    gb = g.astype(jnp.bfloat16)

    acc = lax.dot_general(
        oh4b, b2_ref[...], (((1,), (0,)), ((), ())),
        preferred_element_type=jnp.float32,
    )
    for i in range(NUM_CODEBOOKS):
        gi = gb * oh4b[:, i:i + 1]
        acc = acc + lax.dot_general(
            gi, w2_ref[i], (((1,), (0,)), ((), ())),
            preferred_element_type=jnp.float32,
        )
    o_ref[...] = acc


def _mlp_tc(e128, ids3d, w1ext, w2, b2):
    return pl.pallas_call(
        _mlp_body,
        grid=(_NT,),
        in_specs=[
            pl.BlockSpec((_TT, 128), lambda i: (i, 0)),
            pl.BlockSpec((1, 1, _TT), lambda i: (i, 0, 0)),
            pl.BlockSpec((NUM_CODEBOOKS * CODEBOOK_DIM + NUM_CODEBOOKS,
                          HIDDEN_SIZE), lambda i: (0, 0)),
            pl.BlockSpec((NUM_CODEBOOKS, HIDDEN_SIZE, HIDDEN_SIZE),
                         lambda i: (0, 0, 0)),
            pl.BlockSpec((NUM_CODEBOOKS, HIDDEN_SIZE), lambda i: (0, 0)),
        ],
        out_specs=pl.BlockSpec((_TT, HIDDEN_SIZE), lambda i: (i, 0)),
        out_shape=jax.ShapeDtypeStruct((T, HIDDEN_SIZE), jnp.float32),
        compiler_params=pltpu.CompilerParams(
            dimension_semantics=("arbitrary",),
        ),
    )(e128, ids3d, w1ext, w2, b2)


def kernel(codec_input_ids, table, W1, b1, W2, b2):
    ids = codec_input_ids.reshape(-1).astype(jnp.int32)
    table128 = table.reshape(-1, 128)
    e128 = _gather_sc()(ids.reshape(_NW * _NCHUNK, _CHUNK), table128)
    w1ext = jnp.concatenate(
        [W1.reshape(NUM_CODEBOOKS * CODEBOOK_DIM, HIDDEN_SIZE), b1], axis=0
    ).astype(jnp.bfloat16)
    out = _mlp_tc(
        e128,
        ids.reshape(_NT, 1, _TT),
        w1ext,
        W2.astype(jnp.bfloat16),
        b2.astype(jnp.bfloat16),
    )
    return out.reshape(B, S, HIDDEN_SIZE)


# speedup vs baseline: 1.0006x; 1.0006x over previous
import functools

import jax
import jax.numpy as jnp
from jax import lax
from jax.experimental import pallas as pl
from jax.experimental.pallas import tpu as pltpu
from jax.experimental.pallas import tpu_sc as plsc

NUM_CODEBOOKS = 4
CODEBOOK_BITS = 17
CODEBOOK_DIM = 16
HIDDEN_SIZE = 768
B, S = 4, 4096
T = B * S
ROWS_PER_128 = 128 // CODEBOOK_DIM

_NC, _NS = 2, 16
_NW = _NC * _NS
_B_PER_W = T // _NW
_CHUNK = 128
_NCHUNK = _B_PER_W // _CHUNK
_L = 16


@functools.cache
def _gather_sc():
    @functools.partial(
        pl.kernel,
        mesh=plsc.VectorSubcoreMesh(core_axis_name="c", subcore_axis_name="s"),
        out_type=jax.ShapeDtypeStruct((T, 128), jnp.float32),
        scratch_types=[
            pltpu.VMEM((_NCHUNK, _CHUNK), jnp.int32),
            pltpu.VMEM((_B_PER_W, 128), jnp.float32),
            pltpu.SemaphoreType.DMA,
        ],
    )
    def gather(ids_hbm, table_hbm, out_hbm, idx_v, rows_v, sem):
        wid = lax.axis_index("s") * _NC + lax.axis_index("c")
        pltpu.sync_copy(ids_hbm.at[pl.ds(wid * _NCHUNK, _NCHUNK)], idx_v)
        for j in range(_NCHUNK):
            for k in range(_CHUNK // _L):
                s = pl.ds(k * _L, _L)
                idx_v[j, s] = lax.shift_right_logical(idx_v[j, s], 3)
        copies = [
            pltpu.async_copy(
                table_hbm.at[idx_v.at[j]],
                rows_v.at[pl.ds(j * _CHUNK, _CHUNK)],
                sem,
            )
            for j in range(_NCHUNK)
        ]
        for cp in copies:
            cp.wait()
        pltpu.sync_copy(rows_v, out_hbm.at[pl.ds(wid * _B_PER_W, _B_PER_W)])

    return gather


_TT = 2048
_NT = T // _TT


def _mlp_body(e_ref, id_ref, w1_ref, w2_ref, b2_ref, o_ref):
    e128 = e_ref[...]
    ids = id_ref[0]
    ids = ids.reshape(_TT, 1)
    sub = jnp.bitwise_and(ids, ROWS_PER_128 - 1)
    c = lax.shift_right_logical(ids, CODEBOOK_BITS)

    oh8 = (sub == lax.broadcasted_iota(jnp.int32, (1, ROWS_PER_128), 1))
    oh8 = oh8.astype(jnp.float32)
    e = None
    for k in range(ROWS_PER_128):
        part = e128[:, k * CODEBOOK_DIM:(k + 1) * CODEBOOK_DIM]
        part = part * oh8[:, k:k + 1]
        e = part if e is None else e + part

    oh4 = (c == lax.broadcasted_iota(jnp.int32, (1, NUM_CODEBOOKS), 1))
    oh4 = oh4.astype(jnp.float32)
    eb = e.astype(jnp.bfloat16)
    oh4b = oh4.astype(jnp.bfloat16)
    placed = jnp.concatenate(
        [eb * oh4b[:, i:i + 1] for i in range(NUM_CODEBOOKS)] + [oh4b],
        axis=1,
    )
    h = lax.dot_general(
        placed, w1_ref[...], (((1,), (0,)), ((), ())),
        preferred_element_type=jnp.float32,
    )

    g = 0.5 * h * (1.0 + lax.erf(h * 0.7071067811865476))
    gb = g.astype(jnp.bfloat16)

    acc = lax.dot_general(
        oh4b, b2_ref[...], (((1,), (0,)), ((), ())),
        preferred_element_type=jnp.float32,
    )
    for i in range(NUM_CODEBOOKS):
        gi = gb * oh4b[:, i:i + 1]
        acc = acc + lax.dot_general(
            gi, w2_ref[i], (((1,), (0,)), ((), ())),
            preferred_element_type=jnp.float32,
        )
    o_ref[...] = acc


def _mlp_tc(e128, ids3d, w1ext, w2, b2):
    return pl.pallas_call(
        _mlp_body,
        grid=(_NT,),
        in_specs=[
            pl.BlockSpec((_TT, 128), lambda i: (i, 0)),
            pl.BlockSpec((1, 1, _TT), lambda i: (i, 0, 0)),
            pl.BlockSpec((NUM_CODEBOOKS * CODEBOOK_DIM + NUM_CODEBOOKS,
                          HIDDEN_SIZE), lambda i: (0, 0)),
            pl.BlockSpec((NUM_CODEBOOKS, HIDDEN_SIZE, HIDDEN_SIZE),
                         lambda i: (0, 0, 0)),
            pl.BlockSpec((NUM_CODEBOOKS, HIDDEN_SIZE), lambda i: (0, 0)),
        ],
        out_specs=pl.BlockSpec((_TT, HIDDEN_SIZE), lambda i: (i, 0)),
        out_shape=jax.ShapeDtypeStruct((T, HIDDEN_SIZE), jnp.float32),
        compiler_params=pltpu.CompilerParams(
            dimension_semantics=("arbitrary",),
        ),
    )(e128, ids3d, w1ext, w2, b2)


def kernel(codec_input_ids, table, W1, b1, W2, b2):
    ids = codec_input_ids.reshape(-1).astype(jnp.int32)
    table128 = table.reshape(-1, 128)
    e128 = _gather_sc()(ids.reshape(_NW * _NCHUNK, _CHUNK), table128)
    w1ext = jnp.concatenate(
        [W1.reshape(NUM_CODEBOOKS * CODEBOOK_DIM, HIDDEN_SIZE), b1], axis=0
    ).astype(jnp.bfloat16)
    out = _mlp_tc(
        e128,
        ids.reshape(_NT, 1, _TT),
        w1ext,
        W2.astype(jnp.bfloat16),
        b2.astype(jnp.bfloat16),
    )
    return out.reshape(B, S, HIDDEN_SIZE)


# speedup vs baseline: 1.0748x; 1.0741x over previous
import functools

import jax
import jax.numpy as jnp
from jax import lax
from jax.experimental import pallas as pl
from jax.experimental.pallas import tpu as pltpu
from jax.experimental.pallas import tpu_sc as plsc

NUM_CODEBOOKS = 4
CODEBOOK_BITS = 17
CODEBOOK_DIM = 16
HIDDEN_SIZE = 768
B, S = 4, 4096
T = B * S

_NC, _NS = 2, 16
_NW = _NC * _NS
_B_PER_W = T // _NW
_CHUNK = 128
_NCHUNK = _B_PER_W // _CHUNK


@functools.cache
def _gather_sc():
    @functools.partial(
        pl.kernel,
        mesh=plsc.VectorSubcoreMesh(core_axis_name="c", subcore_axis_name="s"),
        compiler_params=pltpu.CompilerParams(use_tc_tiling_on_sc=False),
        out_type=jax.ShapeDtypeStruct((T, CODEBOOK_DIM), jnp.float32),
        scratch_types=[
            pltpu.VMEM((_NCHUNK, _CHUNK), jnp.int32),
            pltpu.VMEM((_B_PER_W, CODEBOOK_DIM), jnp.float32),
            pltpu.SemaphoreType.DMA,
        ],
    )
    def gather(ids_hbm, table_hbm, out_hbm, idx_v, rows_v, sem):
        wid = lax.axis_index("s") * _NC + lax.axis_index("c")
        pltpu.sync_copy(ids_hbm.at[pl.ds(wid * _NCHUNK, _NCHUNK)], idx_v)
        copies = [
            pltpu.async_copy(
                table_hbm.at[idx_v.at[j]],
                rows_v.at[pl.ds(j * _CHUNK, _CHUNK)],
                sem,
            )
            for j in range(_NCHUNK)
        ]
        for cp in copies:
            cp.wait()
        pltpu.sync_copy(rows_v, out_hbm.at[pl.ds(wid * _B_PER_W, _B_PER_W)])

    return gather


_TT = 1024
_NT = T // _TT


def _mlp_body(e_ref, id_ref, w1_ref, w2_ref, o_ref):
    e = e_ref[...]
    ids = id_ref[...]
    c = lax.shift_right_logical(ids, CODEBOOK_BITS)

    oh4 = (c == lax.broadcasted_iota(jnp.int32, (1, NUM_CODEBOOKS), 1))
    oh4b = oh4.astype(jnp.bfloat16)
    eb = e.astype(jnp.bfloat16)

    placed = jnp.concatenate(
        [eb * oh4b[:, i:i + 1] for i in range(NUM_CODEBOOKS)] + [oh4b],
        axis=1,
    )
    h = lax.dot_general(
        placed, w1_ref[...], (((1,), (0,)), ((), ())),
        preferred_element_type=jnp.float32,
    )

    g = 0.5 * h * (1.0 + lax.erf(h * 0.7071067811865476))
    gb = g.astype(jnp.bfloat16)

    gwide = jnp.concatenate(
        [gb * oh4b[:, i:i + 1] for i in range(NUM_CODEBOOKS)] + [oh4b],
        axis=1,
    )
    o_ref[...] = lax.dot_general(
        gwide, w2_ref[...], (((1,), (0,)), ((), ())),
        preferred_element_type=jnp.float32,
    )


def _mlp_tc(embeds, ids_col, w1ext, w2ext):
    return pl.pallas_call(
        _mlp_body,
        grid=(_NT,),
        in_specs=[
            pl.BlockSpec((_TT, CODEBOOK_DIM), lambda i: (i, 0)),
            pl.BlockSpec((_TT, 1), lambda i: (i, 0)),
            pl.BlockSpec((NUM_CODEBOOKS * CODEBOOK_DIM + NUM_CODEBOOKS,
                          HIDDEN_SIZE), lambda i: (0, 0)),
            pl.BlockSpec((NUM_CODEBOOKS * (HIDDEN_SIZE + 1),
                          HIDDEN_SIZE), lambda i: (0, 0)),
        ],
        out_specs=pl.BlockSpec((_TT, HIDDEN_SIZE), lambda i: (i, 0)),
        out_shape=jax.ShapeDtypeStruct((T, HIDDEN_SIZE), jnp.float32),
        compiler_params=pltpu.CompilerParams(
            dimension_semantics=("arbitrary",),
        ),
    )(embeds, ids_col, w1ext, w2ext)


def kernel(codec_input_ids, table, W1, b1, W2, b2):
    ids = codec_input_ids.reshape(-1).astype(jnp.int32)
    embeds = _gather_sc()(ids.reshape(_NW * _NCHUNK, _CHUNK), table)
    w1ext = jnp.concatenate(
        [W1.reshape(NUM_CODEBOOKS * CODEBOOK_DIM, HIDDEN_SIZE), b1], axis=0
    ).astype(jnp.bfloat16)
    w2ext = jnp.concatenate(
        [W2.reshape(NUM_CODEBOOKS * HIDDEN_SIZE, HIDDEN_SIZE), b2], axis=0
    ).astype(jnp.bfloat16)
    out = _mlp_tc(embeds, ids.reshape(T, 1), w1ext, w2ext)
    return out.reshape(B, S, HIDDEN_SIZE)
